# baseline (speedup 1.0000x reference)
import numpy as np
import jax
import jax.numpy as jnp
from jax.experimental import pallas as pl
from jax.experimental.pallas import tpu as pltpu

_HID = 768
_DFF = 512
_NE = 8
_NT = 128
_CF = 8
_K1 = _HID * _CF
_K3 = _DFF * _CF
_KB = 1024
_H = np.float32(2.0 / 5.0)
_INV_H = np.float32(2.5)
_SIXTH = np.float32(1.0 / 6.0)
_F32 = jnp.float32
_BF16 = jnp.bfloat16


def _silu(v):
    return v * (1.0 / (1.0 + jnp.exp(-v)))


def _dot(a, b):
    return jnp.dot(a, b, preferred_element_type=_F32)


def _bases_rows(vc, m):
    vr = jnp.repeat(vc, _CF, axis=0)
    rid = jax.lax.broadcasted_iota(jnp.int32, (_CF * m, _NT), 0)
    c = (rid % _CF).astype(_F32)
    t_c = (c - 3.0) * _H - 1.0
    u = (vr - t_c) * _INV_H
    p0 = u * u * u * _SIXTH
    p1 = (((-3.0 * u + 12.0) * u - 12.0) * u + 4.0) * _SIXTH
    p2 = (((3.0 * u - 24.0) * u + 60.0) * u - 44.0) * _SIXTH
    w = 4.0 - u
    p3 = w * w * w * _SIXTH
    zero = jnp.zeros_like(u)
    b = jnp.where((u >= 0.0) & (u < 1.0), p0, zero)
    b = jnp.where((u >= 1.0) & (u < 2.0), p1, b)
    b = jnp.where((u >= 2.0) & (u < 3.0), p2, b)
    b = jnp.where((u >= 3.0) & (u < 4.0), p3, b)
    return b


def _prologue_body(x_ref, gate_ref, sil_ref, spl_ref, w_ref):
    x = x_ref[...]
    sil_ref[...] = _silu(x)
    logits = _dot(gate_ref[...], x)
    eidx = jax.lax.broadcasted_iota(jnp.int32, (_NE, _NT), 0)
    m1 = jnp.max(logits, axis=0, keepdims=True)
    i1 = jnp.min(jnp.where(logits == m1, eidx, _NE), axis=0, keepdims=True)
    masked = jnp.where(eidx == i1, -jnp.inf, logits)
    m2 = jnp.max(masked, axis=0, keepdims=True)
    i2 = jnp.min(jnp.where(masked == m2, eidx, _NE), axis=0, keepdims=True)
    t = jnp.exp(m2 - m1)
    w1 = 1.0 / (1.0 + t)
    w2 = t * w1
    w_ref[...] = jnp.where(eidx == i1, w1, 0.0) + jnp.where(eidx == i2, w2, 0.0)

    def jb(j, carry):
        xc = x_ref[pl.ds(j * 64, 64), :]
        spl_ref[pl.ds(j * 512, 512), :] = _bases_rows(xc, 64).astype(_BF16)
        return carry

    jax.lax.fori_loop(0, _HID // 64, jb, 0)


def _expert_body(
    sil_x, spl_x, b1, s1, b2, s2, b3, s3, out, h1, h2, spl_h
):
    k = pl.program_id(0)

    @pl.when(k == 0)
    def _():
        h1[...] = _dot(b1[...], sil_x[...])
        h2[...] = _dot(b2[...], sil_x[...])

    @pl.when(k < 6)
    def _():
        sp = spl_x[...]
        h1[...] += _dot(s1[...], sp)
        h2[...] += _dot(s2[...], sp)

    @pl.when(k == 6)
    def _():
        h = h1[...] * h2[...]
        h1[...] = h
        out[...] = _dot(b3[...], _silu(h))

        def jb(j, carry):
            hc = h1[pl.ds(j * 64, 64), :]
            spl_h[pl.ds(j * 512, 512), :] = _bases_rows(hc, 64).astype(_BF16)
            return carry

        jax.lax.fori_loop(0, _DFF // 64, jb, 0)

    @pl.when(k >= 6)
    def _():
        j = k - 6
        sp = spl_h[pl.ds(j * _KB, _KB), :]
        out[...] += _dot(s3[...], sp)


def _combine_body(w_ref, *refs):
    fe_refs = refs[:_NE]
    out_ref = refs[_NE]
    acc = w_ref[0:1, :] * fe_refs[0][...]
    for e in range(1, _NE):
        acc = acc + w_ref[e : e + 1, :] * fe_refs[e][...]
    out_ref[...] = acc


def kernel(x, params):
    x_t = x.reshape(-1, _HID).T
    gate = params["gate"]

    vmem = pl.BlockSpec(memory_space=pltpu.VMEM)
    sil_x, spl_x, w_tok = pl.pallas_call(
        _prologue_body,
        out_shape=(
            jax.ShapeDtypeStruct((_HID, _NT), _F32),
            jax.ShapeDtypeStruct((_K1, _NT), _BF16),
            jax.ShapeDtypeStruct((_NE, _NT), _F32),
        ),
        in_specs=[vmem, vmem],
        out_specs=(vmem, vmem, vmem),
    )(x_t, gate)

    expert_call = pl.pallas_call(
        _expert_body,
        grid=(10,),
        out_shape=jax.ShapeDtypeStruct((_HID, _NT), _F32),
        in_specs=[
            pl.BlockSpec((_HID, _NT), lambda k: (0, 0)),
            pl.BlockSpec((_KB, _NT), lambda k: (jnp.minimum(k, 5), 0)),
            pl.BlockSpec((_DFF, _HID), lambda k: (0, 0)),
            pl.BlockSpec((_DFF, _KB), lambda k: (0, jnp.minimum(k, 5))),
            pl.BlockSpec((_DFF, _HID), lambda k: (0, 0)),
            pl.BlockSpec((_DFF, _KB), lambda k: (0, jnp.minimum(k, 5))),
            pl.BlockSpec((_HID, _DFF), lambda k: (0, 0)),
            pl.BlockSpec(
                (_HID, _KB), lambda k: (0, jnp.clip(k - 6, 0, 3))
            ),
        ],
        out_specs=pl.BlockSpec((_HID, _NT), lambda k: (0, 0)),
        scratch_shapes=[
            pltpu.VMEM((_DFF, _NT), _F32),
            pltpu.VMEM((_DFF, _NT), _F32),
            pltpu.VMEM((_K3, _NT), _BF16),
        ],
    )

    fes = []
    for e in range(_NE):
        p = params["experts"][e]
        args = []
        for name in ("linear1", "linear2", "linear3"):
            q = p[name]
            sw = q["spline_weight"]
            ssc = (sw * q["spline_scaler"][..., None]).astype(_BF16)
            args += [
                q["base_weight"],
                ssc.reshape(sw.shape[0], sw.shape[1] * sw.shape[2]),
            ]
        fes.append(expert_call(sil_x, spl_x, *args))

    out_t = pl.pallas_call(
        _combine_body,
        out_shape=jax.ShapeDtypeStruct((_HID, _NT), _F32),
        in_specs=[vmem] * (1 + _NE),
        out_specs=vmem,
    )(w_tok, *fes)
    return out_t.T.reshape(x.shape)


# speedup vs baseline: 1.1508x; 1.1508x over previous
import numpy as np
import jax
import jax.numpy as jnp
from jax.experimental import pallas as pl
from jax.experimental.pallas import tpu as pltpu

_HID = 768
_DFF = 512
_NE = 8
_NT = 128
_CF = 8
_K1 = _HID * _CF
_K3 = _DFF * _CF
_KB = 1024
_NB3 = _K3 // _KB
_H = np.float32(2.0 / 5.0)
_INV_H = np.float32(2.5)
_SIXTH = np.float32(1.0 / 6.0)
_F32 = jnp.float32


def _silu(v):
    return v * (1.0 / (1.0 + jnp.exp(-v)))


def _dot(a, b):
    return jnp.dot(a, b, preferred_element_type=_F32)


def _bases_rows(vc, m):
    vr = jnp.repeat(vc, _CF, axis=0)
    rid = jax.lax.broadcasted_iota(jnp.int32, (_CF * m, _NT), 0)
    c = (rid % _CF).astype(_F32)
    t_c = (c - 3.0) * _H - 1.0
    u = (vr - t_c) * _INV_H
    p0 = u * u * u * _SIXTH
    p1 = (((-3.0 * u + 12.0) * u - 12.0) * u + 4.0) * _SIXTH
    p2 = (((3.0 * u - 24.0) * u + 60.0) * u - 44.0) * _SIXTH
    w = 4.0 - u
    p3 = w * w * w * _SIXTH
    zero = jnp.zeros_like(u)
    b = jnp.where((u >= 0.0) & (u < 1.0), p0, zero)
    b = jnp.where((u >= 1.0) & (u < 2.0), p1, b)
    b = jnp.where((u >= 2.0) & (u < 3.0), p2, b)
    b = jnp.where((u >= 3.0) & (u < 4.0), p3, b)
    return b


def _crep(c_blk):
    src = jax.lax.broadcasted_iota(jnp.int32, (_NT, _CF * _NT), 0)
    dst = jax.lax.broadcasted_iota(jnp.int32, (_NT, _CF * _NT), 1)
    p = (dst // _CF == src).astype(_F32)
    return _dot(c_blk, p)


def _prologue_body(x_ref, gate_ref, sil_ref, spl_ref, w_ref):
    x = x_ref[...]
    sil_ref[...] = _silu(x)
    logits = _dot(gate_ref[...], x)
    eidx = jax.lax.broadcasted_iota(jnp.int32, (_NE, _NT), 0)
    m1 = jnp.max(logits, axis=0, keepdims=True)
    i1 = jnp.min(jnp.where(logits == m1, eidx, _NE), axis=0, keepdims=True)
    masked = jnp.where(eidx == i1, -jnp.inf, logits)
    m2 = jnp.max(masked, axis=0, keepdims=True)
    i2 = jnp.min(jnp.where(masked == m2, eidx, _NE), axis=0, keepdims=True)
    t = jnp.exp(m2 - m1)
    w1 = 1.0 / (1.0 + t)
    w2 = t * w1
    w_ref[...] = jnp.where(eidx == i1, w1, 0.0) + jnp.where(eidx == i2, w2, 0.0)

    def jb(j, carry):
        xc = x_ref[pl.ds(j * 64, 64), :]
        spl_ref[pl.ds(j * 512, 512), :] = _bases_rows(xc, 64)
        return carry

    jax.lax.fori_loop(0, _HID // 64, jb, 0)


def _expert_body(
    sil_x, spl_x, b1h, c1h, b2h, c2h, b3h, c3h, s1h, s2h, s3h,
    out, h1, h2, spl_h, s1b, s2b, s3b, bb1, cc1, bb2, cc2, bb3, cc3,
    sem1, sem2, sem3, semw,
):
    k = pl.program_id(0)

    def cps(hbm, buf, sem, blk, slot):
        return pltpu.make_async_copy(
            hbm.at[:, pl.ds(blk * _KB, _KB)], buf.at[slot], sem.at[slot]
        )

    cp1 = lambda blk, slot: cps(s1h, s1b, sem1, blk, slot)
    cp2 = lambda blk, slot: cps(s2h, s2b, sem2, blk, slot)
    cp3 = lambda blk, slot: cps(s3h, s3b, sem3, blk, slot)
    smalls = [
        (b1h, bb1), (c1h, cc1), (b2h, bb2), (c2h, cc2), (b3h, bb3), (c3h, cc3)
    ]

    def cpw(i):
        hbm, buf = smalls[i]
        return pltpu.make_async_copy(hbm, buf, semw.at[i])

    slot = jax.lax.rem(k, 2)

    @pl.when(k == 0)
    def _():
        for i in range(6):
            cpw(i).start()
        cp1(0, 0).start()
        cp2(0, 0).start()
        cp3(0, 0).start()
        cpw(0).wait()
        cpw(2).wait()
        h1[...] = _dot(bb1[...], sil_x[...])
        h2[...] = _dot(bb2[...], sil_x[...])
        cpw(1).wait()
        cpw(3).wait()

    @pl.when(k < 6)
    def _():
        cp1(k, slot).wait()
        cp2(k, slot).wait()

        @pl.when(k < 5)
        def _():
            cp1(k + 1, 1 - slot).start()
            cp2(k + 1, 1 - slot).start()

        sp = spl_x[...]
        e1 = _crep(cc1[:, pl.ds(k * _NT, _NT)])
        e2 = _crep(cc2[:, pl.ds(k * _NT, _NT)])
        h1[...] += _dot(s1b[slot] * e1, sp)
        h2[...] += _dot(s2b[slot] * e2, sp)

    @pl.when(k == 6)
    def _():
        cpw(4).wait()
        cpw(5).wait()
        h = h1[...] * h2[...]
        h1[...] = h
        out[...] = _dot(bb3[...], _silu(h))

        def jb(j, carry):
            hc = h1[pl.ds(j * 64, 64), :]
            spl_h[pl.ds(j * 512, 512), :] = _bases_rows(hc, 64)
            return carry

        jax.lax.fori_loop(0, _DFF // 64, jb, 0)

    @pl.when(k >= 6)
    def _():
        j = k - 6
        jslot = jax.lax.rem(j, 2)
        cp3(j, jslot).wait()

        @pl.when(j < _NB3 - 1)
        def _():
            cp3(j + 1, 1 - jslot).start()

        sp = spl_h[pl.ds(j * _KB, _KB), :]
        e3 = _crep(cc3[:, pl.ds(j * _NT, _NT)])
        out[...] += _dot(s3b[jslot] * e3, sp)


def _combine_body(w_ref, *refs):
    fe_refs = refs[:_NE]
    out_ref = refs[_NE]
    acc = w_ref[0:1, :] * fe_refs[0][...]
    for e in range(1, _NE):
        acc = acc + w_ref[e : e + 1, :] * fe_refs[e][...]
    out_ref[...] = acc


def kernel(x, params):
    x_t = x.reshape(-1, _HID).T
    gate = params["gate"]

    vmem = pl.BlockSpec(memory_space=pltpu.VMEM)
    anym = pl.BlockSpec(memory_space=pl.ANY)
    sil_x, spl_x, w_tok = pl.pallas_call(
        _prologue_body,
        out_shape=(
            jax.ShapeDtypeStruct((_HID, _NT), _F32),
            jax.ShapeDtypeStruct((_K1, _NT), _F32),
            jax.ShapeDtypeStruct((_NE, _NT), _F32),
        ),
        in_specs=[vmem, vmem],
        out_specs=(vmem, vmem, vmem),
    )(x_t, gate)

    expert_call = pl.pallas_call(
        _expert_body,
        grid=(10,),
        out_shape=jax.ShapeDtypeStruct((_HID, _NT), _F32),
        in_specs=[
            pl.BlockSpec((_HID, _NT), lambda k: (0, 0)),
            pl.BlockSpec((_KB, _NT), lambda k: (jnp.minimum(k, 5), 0)),
        ]
        + [anym] * 9,
        out_specs=pl.BlockSpec((_HID, _NT), lambda k: (0, 0)),
        scratch_shapes=[
            pltpu.VMEM((_DFF, _NT), _F32),
            pltpu.VMEM((_DFF, _NT), _F32),
            pltpu.VMEM((_K3, _NT), _F32),
            pltpu.VMEM((2, _DFF, _KB), _F32),
            pltpu.VMEM((2, _DFF, _KB), _F32),
            pltpu.VMEM((2, _HID, _KB), _F32),
            pltpu.VMEM((_DFF, _HID), _F32),
            pltpu.VMEM((_DFF, _HID), _F32),
            pltpu.VMEM((_DFF, _HID), _F32),
            pltpu.VMEM((_DFF, _HID), _F32),
            pltpu.VMEM((_HID, _DFF), _F32),
            pltpu.VMEM((_HID, _DFF), _F32),
            pltpu.SemaphoreType.DMA((2,)),
            pltpu.SemaphoreType.DMA((2,)),
            pltpu.SemaphoreType.DMA((2,)),
            pltpu.SemaphoreType.DMA((6,)),
        ],
    )

    fes = []
    for e in range(_NE):
        p = params["experts"][e]
        bases = []
        spls = []
        for name in ("linear1", "linear2", "linear3"):
            q = p[name]
            sw = q["spline_weight"]
            bases += [q["base_weight"], q["spline_scaler"]]
            spls.append(sw.reshape(sw.shape[0], sw.shape[1] * sw.shape[2]))
        fes.append(expert_call(sil_x, spl_x, *bases, *spls))

    out_t = pl.pallas_call(
        _combine_body,
        out_shape=jax.ShapeDtypeStruct((_HID, _NT), _F32),
        in_specs=[vmem] * (1 + _NE),
        out_specs=vmem,
    )(w_tok, *fes)
    return out_t.T.reshape(x.shape)
